# Initial kernel scaffold; baseline (speedup 1.0000x reference)
import jax
import jax.numpy as jnp
from jax.experimental import pallas as pl


def kernel(x, edge_index, W1a, b1a, W2a, b2a, W1b, b1b, W2b, b2b):
    raise NotImplementedError("write your pallas kernel here")


# speedup vs baseline: 12.6826x; 12.6826x over previous
import functools

import jax
import jax.numpy as jnp
from jax import lax
from jax.experimental import pallas as pl
from jax.experimental.pallas import tpu as pltpu
from jax.experimental.pallas import tpu_sc as plsc

N_NODES = 10000
IN_CH = 128
HID = 16
OUT_CH = 128
E = 320000

NC, NS, LANES = 2, 16, 16
NW = NC * NS
E_W = E // NW
BLK = 128
NBLK = -(-E_W // BLK)
E_W_PAD = NBLK * BLK
PAD_ROWS = 112
NPAD = N_NODES + PAD_ROWS
ROWS_PER_SUB = NPAD // NS


def _seg_sum_body(y_hbm, src_hbm, dst_hbm, out_hbm,
                  src_v, dst_v, rows_v, zrow_v, acc_sh, sem):
    c = lax.axis_index("c")
    s = lax.axis_index("s")
    wid = s * NC + c

    pltpu.sync_copy(src_hbm.at[wid], src_v)
    pltpu.sync_copy(dst_hbm.at[wid], dst_v)

    def zbody(i, carry):
        zrow_v[i, :] = jnp.zeros((LANES,), jnp.float32)
        return carry
    lax.fori_loop(0, ROWS_PER_SUB, zbody, 0)
    pltpu.sync_copy(zrow_v, acc_sh.at[pl.ds(s * ROWS_PER_SUB, ROWS_PER_SUB)])
    plsc.subcore_barrier()

    def ebody(j, carry):
        pltpu.async_copy(y_hbm.at[src_v.at[j]], rows_v, sem).wait()
        pltpu.sync_copy(rows_v, acc_sh.at[dst_v.at[j]], add=True)
        return carry
    lax.fori_loop(0, NBLK, ebody, 0)
    plsc.subcore_barrier()

    pltpu.sync_copy(acc_sh.at[pl.ds(s * ROWS_PER_SUB, ROWS_PER_SUB)],
                    out_hbm.at[c, pl.ds(s * ROWS_PER_SUB, ROWS_PER_SUB)])


def _seg_sum(y, srcw, dstw):
    mesh = plsc.VectorSubcoreMesh(core_axis_name="c", subcore_axis_name="s",
                                  num_cores=NC, num_subcores=NS)
    return pl.kernel(
        _seg_sum_body,
        out_type=jax.ShapeDtypeStruct((NC, NPAD, LANES), jnp.float32),
        mesh=mesh,
        scratch_types=[
            pltpu.VMEM((NBLK, BLK), jnp.int32),
            pltpu.VMEM((NBLK, BLK), jnp.int32),
            pltpu.VMEM((BLK, LANES), jnp.float32),
            pltpu.VMEM((ROWS_PER_SUB, LANES), jnp.float32),
            pltpu.VMEM_SHARED((NPAD, LANES), jnp.float32),
            pltpu.SemaphoreType.DMA,
        ],
        compiler_params=pltpu.CompilerParams(use_tc_tiling_on_sc=False),
    )(y, srcw, dstw)


def _mm1(x, W1a):
    def body(x_ref, w_ref, o_ref):
        o_ref[...] = jnp.dot(x_ref[...], w_ref[...],
                             preferred_element_type=jnp.float32)
    return pl.pallas_call(
        body,
        out_shape=jax.ShapeDtypeStruct((N_NODES, HID), jnp.float32),
    )(x, W1a)


def _mid(y, parts, b1a, W2a, b2a):
    def body(y_ref, p_ref, b1_ref, w2_ref, b2_ref, o_ref):
        agg = p_ref[0, :N_NODES, :] + p_ref[1, :N_NODES, :]
        u = jnp.maximum(y_ref[...] + agg + b1_ref[...], 0.0)
        v = jnp.dot(u, w2_ref[...], preferred_element_type=jnp.float32)
        o_ref[...] = jnp.maximum(v + b2_ref[...], 0.0)
    return pl.pallas_call(
        body,
        out_shape=jax.ShapeDtypeStruct((N_NODES, HID), jnp.float32),
    )(y, parts, b1a, W2a, b2a)


def _final(h1, parts, W1b, b1b, W2b, b2b):
    def body(h_ref, p_ref, w1_ref, b1_ref, w2_ref, b2_ref, o_ref):
        agg = p_ref[0, :N_NODES, :] + p_ref[1, :N_NODES, :]
        g = h_ref[...] + agg
        t = jnp.dot(g, w1_ref[...], preferred_element_type=jnp.float32)
        t = jnp.maximum(t + b1_ref[...], 0.0)
        o_ref[...] = jnp.dot(t, w2_ref[...],
                             preferred_element_type=jnp.float32) + b2_ref[...]
    return pl.pallas_call(
        body,
        out_shape=jax.ShapeDtypeStruct((N_NODES, OUT_CH), jnp.float32),
    )(h1, parts, W1b, b1b, W2b, b2b)


def kernel(x, edge_index, W1a, b1a, W2a, b2a, W1b, b1b, W2b, b2b):
    ei = edge_index.astype(jnp.int32)
    pad = E_W_PAD - E_W
    srcw = jnp.pad(ei[0].reshape(NW, E_W), ((0, 0), (0, pad)))
    pad_dst = N_NODES + (jnp.arange(pad, dtype=jnp.int32) % PAD_ROWS)
    dstw = jnp.concatenate(
        [ei[1].reshape(NW, E_W), jnp.broadcast_to(pad_dst, (NW, pad))], axis=1)
    srcw = srcw.reshape(NW, NBLK, BLK)
    dstw = dstw.reshape(NW, NBLK, BLK)

    y = _mm1(x, W1a)
    p1 = _seg_sum(y, srcw, dstw)
    h1 = _mid(y, p1, b1a.reshape(1, HID), W2a, b2a.reshape(1, HID))
    p2 = _seg_sum(h1, srcw, dstw)
    return _final(h1, p2, W1b, b1b.reshape(1, OUT_CH), W2b,
                  b2b.reshape(1, OUT_CH))


# speedup vs baseline: 24.4856x; 1.9306x over previous
import functools

import jax
import jax.numpy as jnp
from jax import lax
from jax.experimental import pallas as pl
from jax.experimental.pallas import tpu as pltpu
from jax.experimental.pallas import tpu_sc as plsc

N_NODES = 10000
IN_CH = 128
HID = 16
OUT_CH = 128
E = 320000

NC, NS, LANES = 2, 16, 16
NW = NC * NS
E_W = E // NW
BLK = 128
NBLK = 80
E_W_PAD = NBLK * BLK
PAD_ROWS = 112
NPAD = N_NODES + PAD_ROWS
ROWS_PER_SUB = NPAD // NS


ROWS_STAGE = N_NODES // NS


def _seg_sum_body(y_hbm, src_hbm, dst_hbm, out_hbm,
                  src_v, dst_v, bufa, bufb, zrow_v, y_sh, acc_sh,
                  sema, semb):
    c = lax.axis_index("c")
    s = lax.axis_index("s")
    wid = s * NC + c

    pltpu.sync_copy(src_hbm.at[wid], src_v)
    pltpu.sync_copy(dst_hbm.at[wid], dst_v)
    pltpu.sync_copy(y_hbm.at[pl.ds(s * ROWS_STAGE, ROWS_STAGE)],
                    y_sh.at[pl.ds(s * ROWS_STAGE, ROWS_STAGE)])

    def zbody(i, carry):
        zrow_v[i, :] = jnp.zeros((LANES,), jnp.float32)
        return carry
    lax.fori_loop(0, ROWS_PER_SUB, zbody, 0)
    pltpu.sync_copy(zrow_v, acc_sh.at[pl.ds(s * ROWS_PER_SUB, ROWS_PER_SUB)])
    plsc.subcore_barrier()

    pltpu.async_copy(y_sh.at[src_v.at[0]], bufa, sema)

    def ebody(jj, carry):
        j0 = 2 * jj
        pltpu.async_copy(y_sh.at[src_v.at[j0 + 1]], bufb, semb)
        pltpu.make_async_copy(y_sh.at[src_v.at[j0]], bufa, sema).wait()
        pltpu.sync_copy(bufa, acc_sh.at[dst_v.at[j0]], add=True)

        @pl.when(jj + 1 < NBLK // 2)
        def _():
            pltpu.async_copy(y_sh.at[src_v.at[j0 + 2]], bufa, sema)

        pltpu.make_async_copy(y_sh.at[src_v.at[j0 + 1]], bufb, semb).wait()
        pltpu.sync_copy(bufb, acc_sh.at[dst_v.at[j0 + 1]], add=True)
        return carry
    lax.fori_loop(0, NBLK // 2, ebody, 0)
    plsc.subcore_barrier()

    pltpu.sync_copy(acc_sh.at[pl.ds(s * ROWS_PER_SUB, ROWS_PER_SUB)],
                    out_hbm.at[c, pl.ds(s * ROWS_PER_SUB, ROWS_PER_SUB)])


def _seg_sum(y, srcw, dstw):
    mesh = plsc.VectorSubcoreMesh(core_axis_name="c", subcore_axis_name="s",
                                  num_cores=NC, num_subcores=NS)
    return pl.kernel(
        _seg_sum_body,
        out_type=jax.ShapeDtypeStruct((NC, NPAD, LANES), jnp.float32),
        mesh=mesh,
        scratch_types=[
            pltpu.VMEM((NBLK, BLK), jnp.int32),
            pltpu.VMEM((NBLK, BLK), jnp.int32),
            pltpu.VMEM((BLK, LANES), jnp.float32),
            pltpu.VMEM((BLK, LANES), jnp.float32),
            pltpu.VMEM((ROWS_PER_SUB, LANES), jnp.float32),
            pltpu.VMEM_SHARED((N_NODES, LANES), jnp.float32),
            pltpu.VMEM_SHARED((NPAD, LANES), jnp.float32),
            pltpu.SemaphoreType.DMA,
            pltpu.SemaphoreType.DMA,
        ],
        compiler_params=pltpu.CompilerParams(use_tc_tiling_on_sc=False),
    )(y, srcw, dstw)


def _mm1(x, W1a):
    def body(x_ref, w_ref, o_ref):
        o_ref[...] = jnp.dot(x_ref[...], w_ref[...],
                             preferred_element_type=jnp.float32)
    return pl.pallas_call(
        body,
        out_shape=jax.ShapeDtypeStruct((N_NODES, HID), jnp.float32),
    )(x, W1a)


def _mid(y, parts, b1a, W2a, b2a):
    def body(y_ref, p_ref, b1_ref, w2_ref, b2_ref, o_ref):
        agg = p_ref[0, :N_NODES, :] + p_ref[1, :N_NODES, :]
        u = jnp.maximum(y_ref[...] + agg + b1_ref[...], 0.0)
        v = jnp.dot(u, w2_ref[...], preferred_element_type=jnp.float32)
        o_ref[...] = jnp.maximum(v + b2_ref[...], 0.0)
    return pl.pallas_call(
        body,
        out_shape=jax.ShapeDtypeStruct((N_NODES, HID), jnp.float32),
    )(y, parts, b1a, W2a, b2a)


def _final(h1, parts, W1b, b1b, W2b, b2b):
    def body(h_ref, p_ref, w1_ref, b1_ref, w2_ref, b2_ref, o_ref):
        agg = p_ref[0, :N_NODES, :] + p_ref[1, :N_NODES, :]
        g = h_ref[...] + agg
        t = jnp.dot(g, w1_ref[...], preferred_element_type=jnp.float32)
        t = jnp.maximum(t + b1_ref[...], 0.0)
        o_ref[...] = jnp.dot(t, w2_ref[...],
                             preferred_element_type=jnp.float32) + b2_ref[...]
    return pl.pallas_call(
        body,
        out_shape=jax.ShapeDtypeStruct((N_NODES, OUT_CH), jnp.float32),
    )(h1, parts, W1b, b1b, W2b, b2b)


def kernel(x, edge_index, W1a, b1a, W2a, b2a, W1b, b1b, W2b, b2b):
    ei = edge_index.astype(jnp.int32)
    pad = E_W_PAD - E_W
    srcw = jnp.pad(ei[0].reshape(NW, E_W), ((0, 0), (0, pad)))
    pad_dst = N_NODES + (jnp.arange(pad, dtype=jnp.int32) % PAD_ROWS)
    dstw = jnp.concatenate(
        [ei[1].reshape(NW, E_W), jnp.broadcast_to(pad_dst, (NW, pad))], axis=1)
    srcw = srcw.reshape(NW, NBLK, BLK)
    dstw = dstw.reshape(NW, NBLK, BLK)

    y = _mm1(x, W1a)
    p1 = _seg_sum(y, srcw, dstw)
    h1 = _mid(y, p1, b1a.reshape(1, HID), W2a, b2a.reshape(1, HID))
    p2 = _seg_sum(h1, srcw, dstw)
    return _final(h1, p2, W1b, b1b.reshape(1, OUT_CH), W2b,
                  b2b.reshape(1, OUT_CH))
